# Initial kernel scaffold; baseline (speedup 1.0000x reference)
import jax
import jax.numpy as jnp
from jax.experimental import pallas as pl


def kernel(x, edge_index, batch, context, proj_query, proj_keys, proj_values, query_coef, proj_final):
    raise NotImplementedError("write your pallas kernel here")


# speedup vs baseline: 19.7351x; 19.7351x over previous
import math
import functools

import jax
import jax.numpy as jnp
from jax.experimental import pallas as pl
from jax.experimental.pallas import tpu as pltpu

N = 10000
B = 64
H = 8
DV = 16
DC = 128
DE = 124
HD = H * DV

TN = 1000
T = N // TN
NEG = -1e30


def _body(x_ref, b_ref, wk_ref, wv_ref, ctx_ref, wq_ref, qc_ref, pf_ref,
          out_ref, u_s, segmax_s, segsum_s, agg_s, qflat_s):
    i = pl.program_id(0)
    t = jax.lax.rem(i, T)

    lane = jax.lax.broadcasted_iota(jnp.int32, (HD, H), 0)
    head = jax.lax.broadcasted_iota(jnp.int32, (HD, H), 1)
    S = (lane // DV == head).astype(jnp.float32)
    lane2 = jax.lax.broadcasted_iota(jnp.int32, (HD, DV), 0)
    vpos = jax.lax.broadcasted_iota(jnp.int32, (HD, DV), 1)
    R = (jax.lax.rem(lane2, DV) == vpos).astype(jnp.float32)

    @pl.when(i == 0)
    def _init():
        qflat_s[:, :] = jnp.dot(ctx_ref[:, :], wq_ref[:, :],
                                preferred_element_type=jnp.float32)
        segmax_s[:, :] = jnp.full((B, H), NEG, jnp.float32)
        segsum_s[:, :] = jnp.zeros((B, H), jnp.float32)
        agg_s[:, :] = jnp.zeros((B, HD), jnp.float32)

    batch_col = b_ref[:, 0:1]
    iota_b = jax.lax.broadcasted_iota(jnp.int32, (TN, B), 1)
    maskb = batch_col == iota_b
    maskf = maskb.astype(jnp.float32)

    x = x_ref[:, :]

    @pl.when(i < T)
    def _phase1():
        k = jnp.dot(x, wk_ref[:, :], preferred_element_type=jnp.float32)
        qg = jnp.dot(maskf, qflat_s[:, :],
                     preferred_element_type=jnp.float32)
        u = jnp.dot(k * qg, S,
                    preferred_element_type=jnp.float32) * (1.0 / math.sqrt(DV))
        u_s[pl.ds(t * TN, TN), :] = u
        for h in range(H):
            col = u[:, h:h + 1]
            masked = jnp.where(maskb, col, NEG)
            tmax = jnp.max(masked, axis=0)
            cur = segmax_s[:, h]
            segmax_s[:, h] = jnp.maximum(cur, tmax)

    @pl.when(i >= T)
    def _phase2():
        v = jnp.dot(x, wv_ref[:, :], preferred_element_type=jnp.float32)
        u = u_s[pl.ds(t * TN, TN), :]
        gmax = jnp.dot(maskf, segmax_s[:, :],
                       preferred_element_type=jnp.float32)
        e = jnp.exp(u - gmax)
        e_exp = jnp.dot(e, S.T, preferred_element_type=jnp.float32)
        w = e_exp * v
        contract0 = (((0,), (0,)), ((), ()))
        agg_s[:, :] += jax.lax.dot_general(
            maskf, w, contract0, preferred_element_type=jnp.float32)
        segsum_s[:, :] += jax.lax.dot_general(
            maskf, e, contract0, preferred_element_type=jnp.float32)

        @pl.when(i == 2 * T - 1)
        def _final():
            ssum_exp = jnp.dot(segsum_s[:, :], S.T,
                               preferred_element_type=jnp.float32)
            agg = agg_s[:, :] / (ssum_exp + 1e-16)
            hf = qc_ref[0, 0] * qflat_s[:, :] + agg
            hsum = jnp.dot(hf, R, preferred_element_type=jnp.float32)
            out_ref[:, :] = jnp.dot(hsum, pf_ref[:, :],
                                    preferred_element_type=jnp.float32)


@jax.jit
def _run(x, batch2d, wk, wv, context, wq, qcb, pf):
    grid = (2 * T,)
    return pl.pallas_call(
        _body,
        grid=grid,
        in_specs=[
            pl.BlockSpec((TN, DC), lambda i: (jax.lax.rem(i, T), 0)),
            pl.BlockSpec((TN, 1), lambda i: (jax.lax.rem(i, T), 0)),
            pl.BlockSpec((DC, HD), lambda i: (0, 0)),
            pl.BlockSpec((DC, HD), lambda i: (0, 0)),
            pl.BlockSpec((B, DC), lambda i: (0, 0)),
            pl.BlockSpec((DC, HD), lambda i: (0, 0)),
            pl.BlockSpec((8, 128), lambda i: (0, 0)),
            pl.BlockSpec((DV, DE), lambda i: (0, 0)),
        ],
        out_specs=pl.BlockSpec((B, DE), lambda i: (0, 0)),
        out_shape=jax.ShapeDtypeStruct((B, DE), jnp.float32),
        scratch_shapes=[
            pltpu.VMEM((N, H), jnp.float32),
            pltpu.VMEM((B, H), jnp.float32),
            pltpu.VMEM((B, H), jnp.float32),
            pltpu.VMEM((B, HD), jnp.float32),
            pltpu.VMEM((B, HD), jnp.float32),
        ],
    )(x, batch2d, wk, wv, context, wq, qcb, pf)


def kernel(x, edge_index, batch, context, proj_query, proj_keys, proj_values,
           query_coef, proj_final):
    wk = proj_keys.transpose(1, 0, 2).reshape(DC, HD)
    wv = proj_values.transpose(1, 0, 2).reshape(DC, HD)
    wq = proj_query.transpose(1, 0, 2).reshape(DC, HD)
    batch2d = batch.reshape(N, 1)
    qcb = jnp.broadcast_to(query_coef.reshape(1, 1), (8, 128))
    out = _run(x, batch2d, wk, wv, context, wq, qcb, proj_final)
    return out
